# baseline (speedup 1.0000x reference)
import functools

import numpy as np
import jax
import jax.numpy as jnp
from jax import lax
from jax.experimental import pallas as pl
from jax.experimental.pallas import tpu as pltpu
from jax.experimental.pallas import tpu_sc as plsc

_OC, _IC, _ORD, _K = 384, 192, 4, 5
_ROW = _ORD * _K * _K
_BLK = _IC * _ROW
_NW = 32
_OC_PER_W = _OC // _NW
_LANES = 16
_QUAD = 4 * _ROW
_VPQ = _QUAD // _LANES


def _perm_tables() -> np.ndarray:
    a = np.arange(_ROW).reshape(_ORD, _K, _K)
    tabs = []
    for r in (1, 2, 3):
        p = np.rot90(np.roll(a, shift=r, axis=0), k=r, axes=(-2, -1)).reshape(_ROW)
        tabs.append(np.concatenate([p + q * _ROW for q in range(4)]))
    return np.concatenate(tabs).astype(np.int32)


_IDX_TAB = _perm_tables()

_MESH = plsc.VectorSubcoreMesh(core_axis_name="c", subcore_axis_name="s",
                               num_cores=2, num_subcores=16)


@functools.partial(
    pl.kernel,
    out_type=jax.ShapeDtypeStruct((_OC * _ORD * _BLK,), jnp.float32),
    mesh=_MESH,
    scratch_types=[
        pltpu.VMEM((3 * _QUAD,), jnp.int32),
        pltpu.VMEM((_BLK,), jnp.float32),
        pltpu.VMEM((2, _BLK), jnp.float32),
        pltpu.SemaphoreType.DMA,
        pltpu.SemaphoreType.DMA,
        pltpu.SemaphoreType.DMA,
    ],
    compiler_params=pltpu.CompilerParams(needs_layout_passes=False),
)
def _bank(w_hbm, idx_hbm, out_hbm, idx_v, in_v, out_v, sem0, sem_a, sem_b):
    wid = lax.axis_index("s") * 2 + lax.axis_index("c")
    pltpu.sync_copy(idx_hbm, idx_v)
    out_sems = (sem_a, sem_b)

    def per_oc(t, carry):
        oc = wid * _OC_PER_W + t
        pltpu.sync_copy(w_hbm.at[pl.ds(oc * _BLK, _BLK)], in_v)
        c0 = pltpu.async_copy(
            in_v, out_hbm.at[pl.ds(oc * _ORD * _BLK, _BLK)], sem0)
        copies = []
        for r in range(3):
            b = r & 1
            idx0 = tuple(
                idx_v[pl.ds(r * _QUAD + v * _LANES, _LANES)] for v in range(_VPQ))

            def body(i, idx, b=b):
                for v in range(_VPQ):
                    vals = plsc.load_gather(in_v, [idx[v]])
                    out_v[b, pl.ds(i + v * _LANES, _LANES)] = vals
                return tuple(x + _QUAD for x in idx)

            if r == 2:
                copies[0].wait()
            plsc.parallel_loop(0, _BLK, step=_QUAD, unroll=2, carry=idx0)(body)
            copies.append(pltpu.async_copy(
                out_v.at[b],
                out_hbm.at[pl.ds((oc * _ORD + r + 1) * _BLK, _BLK)],
                out_sems[b]))
        c0.wait()
        copies[1].wait()
        copies[2].wait()
        return carry

    lax.fori_loop(0, _OC_PER_W, per_oc, 0)


def kernel(weight):
    w1 = weight.reshape(_OC * _IC * _ROW)
    out = _bank(w1, jnp.asarray(_IDX_TAB))
    return out.reshape(_OC, _ORD, _IC, _ORD, _K, _K)


# speedup vs baseline: 6.0246x; 6.0246x over previous
import functools

import numpy as np
import jax
import jax.numpy as jnp
from jax import lax
from jax.experimental import pallas as pl
from jax.experimental.pallas import tpu as pltpu
from jax.experimental.pallas import tpu_sc as plsc

_OC, _IC, _ORD, _K = 384, 192, 4, 5
_ROW = _ORD * _K * _K
_NSUB_IN = _IC * _K * _K * 3 * _ORD
_NSUB_OUT = 4 * _NSUB_IN
_NW = 32
_SPW = _NSUB_OUT // _NW
_CH = 120
_NCH = _SPW // _CH


def _subrow_table() -> np.ndarray:
    a = np.arange(_ROW).reshape(_ORD, _K, _K)
    perms = [np.rot90(np.roll(a, shift=r, axis=0), k=r, axes=(-2, -1)).reshape(_ROW)
             for r in range(4)]
    r_, ic_, y_, x_, t_, h_ = np.meshgrid(
        np.arange(4), np.arange(_IC), np.arange(_K), np.arange(_K),
        np.arange(3), np.arange(_ORD), indexing="ij")
    j = (h_ * _K + y_) * _K + x_
    src = np.stack(perms)[r_.ravel(), j.ravel()].reshape(j.shape)
    hp = src // (_K * _K)
    yp = (src % (_K * _K)) // _K
    xp = src % _K
    tab = ((((ic_ * _K + yp) * _K + xp) * 3 + t_) * 4 + hp).reshape(-1)
    return tab.astype(np.int32).reshape(_NW, _NCH, _CH)


_TAB = _subrow_table()

_MESH = plsc.VectorSubcoreMesh(core_axis_name="c", subcore_axis_name="s",
                               num_cores=2, num_subcores=16)


@functools.partial(
    pl.kernel,
    out_type=jax.ShapeDtypeStruct((_NSUB_OUT, 128), jnp.float32),
    mesh=_MESH,
    scratch_types=[
        pltpu.VMEM((_NCH, _CH), jnp.int32),
        pltpu.VMEM((2, _CH, 128), jnp.float32),
        pltpu.SemaphoreType.DMA,
        pltpu.SemaphoreType.DMA,
        pltpu.SemaphoreType.DMA,
        pltpu.SemaphoreType.DMA,
    ],
    compiler_params=pltpu.CompilerParams(needs_layout_passes=False),
)
def _bank(in_hbm, tab_hbm, out_hbm, idx_v, buf, gs0, gs1, os0, os1):
    wid = lax.axis_index("s") * 2 + lax.axis_index("c")
    base = wid * _SPW
    pltpu.sync_copy(tab_hbm.at[wid], idx_v)
    gsems = (gs0, gs1)
    osems = (os0, os1)
    out_handles = []
    for c in range(_NCH):
        b = c & 1
        if c >= 2:
            out_handles[c - 2].wait()
        pltpu.async_copy(in_hbm.at[idx_v.at[c]], buf.at[b], gsems[b]).wait()
        out_handles.append(pltpu.async_copy(
            buf.at[b], out_hbm.at[pl.ds(base + c * _CH, _CH)], osems[b]))
    out_handles[-2].wait()
    out_handles[-1].wait()


def kernel(weight):
    in2 = (weight.reshape(3, 128, _IC, _ORD, _K, _K)
           .transpose(2, 4, 5, 0, 3, 1).reshape(_NSUB_IN, 128))
    out2 = _bank(in2, jnp.asarray(_TAB))
    out7 = out2.reshape(4, _IC, _K, _K, 3, _ORD, 128)
    return out7.transpose(4, 6, 0, 1, 5, 2, 3).reshape(_OC, 4, _IC, _ORD, _K, _K)


# speedup vs baseline: 15.0232x; 2.4936x over previous
import functools

import numpy as np
import jax
import jax.numpy as jnp
from jax import lax
from jax.experimental import pallas as pl
from jax.experimental.pallas import tpu as pltpu
from jax.experimental.pallas import tpu_sc as plsc

_OC, _IC, _ORD, _K = 384, 192, 4, 5
_ROW = _ORD * _K * _K
_NR_IN = _IC * _K * _K * _ORD
_NR_OUT = 4 * _NR_IN
_NW = 32
_RPW = _NR_OUT // _NW
_CH = 120
_NCH = _RPW // _CH


def _row_table() -> np.ndarray:
    a = np.arange(_ROW).reshape(_ORD, _K, _K)
    perms = [np.rot90(np.roll(a, shift=r, axis=0), k=r, axes=(-2, -1)).reshape(_ROW)
             for r in range(4)]
    r_, ic_, y_, x_, h_ = np.meshgrid(
        np.arange(4), np.arange(_IC), np.arange(_K), np.arange(_K),
        np.arange(_ORD), indexing="ij")
    j = (h_ * _K + y_) * _K + x_
    src = np.stack(perms)[r_.ravel(), j.ravel()].reshape(j.shape)
    hp = src // (_K * _K)
    yp = (src % (_K * _K)) // _K
    xp = src % _K
    tab = (((ic_ * _K + yp) * _K + xp) * _ORD + hp).reshape(-1)
    return tab.astype(np.int32).reshape(_NW, _NCH, _CH)


_TAB = _row_table()

_MESH = plsc.VectorSubcoreMesh(core_axis_name="c", subcore_axis_name="s",
                               num_cores=2, num_subcores=16)


@functools.partial(
    pl.kernel,
    out_type=jax.ShapeDtypeStruct((_NR_OUT, _OC), jnp.float32),
    mesh=_MESH,
    scratch_types=[
        pltpu.VMEM((_NCH, _CH), jnp.int32),
        pltpu.VMEM((2, _CH, _OC), jnp.float32),
        pltpu.SemaphoreType.DMA,
        pltpu.SemaphoreType.DMA,
        pltpu.SemaphoreType.DMA,
        pltpu.SemaphoreType.DMA,
    ],
    compiler_params=pltpu.CompilerParams(needs_layout_passes=False),
)
def _bank(in_hbm, tab_hbm, out_hbm, idx_v, buf, gs0, gs1, os0, os1):
    wid = lax.axis_index("s") * 2 + lax.axis_index("c")
    base = wid * _RPW
    pltpu.sync_copy(tab_hbm.at[wid], idx_v)
    gsems = (gs0, gs1)
    osems = (os0, os1)
    out_handles = []
    for c in range(_NCH):
        b = c & 1
        if c >= 2:
            out_handles[c - 2].wait()
        pltpu.async_copy(in_hbm.at[idx_v.at[c]], buf.at[b], gsems[b]).wait()
        out_handles.append(pltpu.async_copy(
            buf.at[b], out_hbm.at[pl.ds(base + c * _CH, _CH)], osems[b]))
    out_handles[-2].wait()
    out_handles[-1].wait()


def kernel(weight):
    in2 = weight.transpose(1, 3, 4, 2, 0).reshape(_NR_IN, _OC)
    out2 = _bank(in2, jnp.asarray(_TAB))
    out6 = out2.reshape(4, _IC, _K, _K, _ORD, _OC)
    return out6.transpose(5, 0, 1, 4, 2, 3)


# speedup vs baseline: 26.7555x; 1.7809x over previous
import functools

import numpy as np
import jax
import jax.numpy as jnp
from jax import lax
from jax.experimental import pallas as pl
from jax.experimental.pallas import tpu as pltpu
from jax.experimental.pallas import tpu_sc as plsc

_OC, _IC, _ORD, _K = 384, 192, 4, 5
_ROW = _ORD * _K * _K
_NR_IN = _IC * _K * _K * _ORD
_NR_OUT = 4 * _NR_IN
_NW = 32
_RPW = _NR_OUT // _NW
_CH = 120
_NCH = _RPW // _CH


def _row_table() -> np.ndarray:
    a = np.arange(_ROW).reshape(_ORD, _K, _K)
    perms = [np.rot90(np.roll(a, shift=r, axis=0), k=r, axes=(-2, -1)).reshape(_ROW)
             for r in range(4)]
    r_, ic_, y_, x_, h_ = np.meshgrid(
        np.arange(4), np.arange(_IC), np.arange(_K), np.arange(_K),
        np.arange(_ORD), indexing="ij")
    j = (h_ * _K + y_) * _K + x_
    src = np.stack(perms)[r_.ravel(), j.ravel()].reshape(j.shape)
    hp = src // (_K * _K)
    yp = (src % (_K * _K)) // _K
    xp = src % _K
    tab = (((ic_ * _K + yp) * _K + xp) * _ORD + hp).reshape(-1)
    return tab.astype(np.int32).reshape(_NW, _NCH, _CH)


_TAB = _row_table()

_MESH = plsc.VectorSubcoreMesh(core_axis_name="c", subcore_axis_name="s",
                               num_cores=2, num_subcores=16)


@functools.partial(
    pl.kernel,
    out_type=jax.ShapeDtypeStruct((_NR_OUT // _ORD, _ORD, _OC), jnp.float32),
    mesh=_MESH,
    scratch_types=[
        pltpu.VMEM((_NCH, _CH), jnp.int32),
        pltpu.VMEM((2, _CH, _OC), jnp.float32),
        pltpu.SemaphoreType.DMA,
        pltpu.SemaphoreType.DMA,
        pltpu.SemaphoreType.DMA,
        pltpu.SemaphoreType.DMA,
    ],
    compiler_params=pltpu.CompilerParams(needs_layout_passes=False),
)
def _bank(in_hbm, tab_hbm, out3_hbm, idx_v, buf, gs0, gs1, os0, os1):
    out_hbm = out3_hbm.reshape(_NR_OUT, _OC)
    wid = lax.axis_index("s") * 2 + lax.axis_index("c")
    base = wid * _RPW
    pltpu.sync_copy(tab_hbm.at[wid], idx_v)
    gsems = (gs0, gs1)
    osems = (os0, os1)
    out_handles = []
    for c in range(_NCH):
        b = c & 1
        if c >= 2:
            out_handles[c - 2].wait()
        pltpu.async_copy(in_hbm.at[idx_v.at[c]], buf.at[b], gsems[b]).wait()
        out_handles.append(pltpu.async_copy(
            buf.at[b], out_hbm.at[pl.ds(base + c * _CH, _CH)], osems[b]))
    out_handles[-2].wait()
    out_handles[-1].wait()


def kernel(weight):
    in2 = weight.transpose(1, 3, 4, 2, 0).reshape(_NR_IN, _OC)
    out3 = _bank(in2, jnp.asarray(_TAB))
    out6 = out3.reshape(4, _IC, _K, _K, _ORD, _OC)
    return out6.transpose(5, 0, 1, 4, 2, 3)


# speedup vs baseline: 33.6927x; 1.2593x over previous
import functools

import numpy as np
import jax
import jax.numpy as jnp
from jax import lax
from jax.experimental import pallas as pl
from jax.experimental.pallas import tpu as pltpu
from jax.experimental.pallas import tpu_sc as plsc

_OC, _IC, _ORD, _K = 384, 192, 4, 5
_ROW = _ORD * _K * _K
_NR_IN = _IC * _K * _K * _ORD
_NR_OUT = 4 * _NR_IN
_NW = 32
_RPW = _NR_OUT // _NW
_CH = 120
_NCH = _RPW // _CH


def _row_table() -> np.ndarray:
    a = np.arange(_ROW).reshape(_ORD, _K, _K)
    perms = [np.rot90(np.roll(a, shift=r, axis=0), k=r, axes=(-2, -1)).reshape(_ROW)
             for r in range(4)]
    r_, ic_, y_, x_, h_ = np.meshgrid(
        np.arange(4), np.arange(_IC), np.arange(_K), np.arange(_K),
        np.arange(_ORD), indexing="ij")
    j = (h_ * _K + y_) * _K + x_
    src = np.stack(perms)[r_.ravel(), j.ravel()].reshape(j.shape)
    hp = src // (_K * _K)
    yp = (src % (_K * _K)) // _K
    xp = src % _K
    tab = (((ic_ * _K + yp) * _K + xp) * _ORD + hp).reshape(-1)
    return tab.astype(np.int32).reshape(_NW, _NCH, _CH)


_TAB = _row_table()

_MESH = plsc.VectorSubcoreMesh(core_axis_name="c", subcore_axis_name="s",
                               num_cores=2, num_subcores=16)


@functools.partial(
    pl.kernel,
    out_type=jax.ShapeDtypeStruct((_NR_OUT // _ORD, _ORD, _OC), jnp.float32),
    mesh=_MESH,
    scratch_types=[
        pltpu.VMEM((_NCH, _CH), jnp.int32),
        pltpu.VMEM((2, _CH, _OC), jnp.float32),
        pltpu.SemaphoreType.DMA,
        pltpu.SemaphoreType.DMA,
        pltpu.SemaphoreType.DMA,
        pltpu.SemaphoreType.DMA,
    ],
    compiler_params=pltpu.CompilerParams(needs_layout_passes=False),
)
def _bank(in3_hbm, tab_hbm, out3_hbm, idx_v, buf, gs0, gs1, os0, os1):
    in_hbm = in3_hbm.reshape(_NR_IN, _OC)
    out_hbm = out3_hbm.reshape(_NR_OUT, _OC)
    wid = lax.axis_index("s") * 2 + lax.axis_index("c")
    base = wid * _RPW
    pltpu.sync_copy(tab_hbm.at[wid], idx_v)
    gsems = (gs0, gs1)
    osems = (os0, os1)
    out_handles = []
    for c in range(_NCH):
        b = c & 1
        if c >= 2:
            out_handles[c - 2].wait()
        pltpu.async_copy(in_hbm.at[idx_v.at[c]], buf.at[b], gsems[b]).wait()
        out_handles.append(pltpu.async_copy(
            buf.at[b], out_hbm.at[pl.ds(base + c * _CH, _CH)], osems[b]))
    out_handles[-2].wait()
    out_handles[-1].wait()


def kernel(weight):
    in3 = weight.transpose(1, 3, 4, 2, 0).reshape(_NR_IN // _ORD, _ORD, _OC)
    out3 = _bank(in3, jnp.asarray(_TAB))
    out6 = out3.reshape(4, _IC, _K, _K, _ORD, _OC)
    return out6.transpose(5, 0, 1, 4, 2, 3)


# speedup vs baseline: 36.8844x; 1.0947x over previous
import functools

import numpy as np
import jax
import jax.numpy as jnp
from jax import lax
from jax.experimental import pallas as pl
from jax.experimental.pallas import tpu as pltpu
from jax.experimental.pallas import tpu_sc as plsc

_OC, _IC, _ORD, _K = 384, 192, 4, 5
_ROW = _ORD * _K * _K
_NR_IN = _IC * _K * _K * _ORD
_NR_OUT = 4 * _NR_IN
_NW = 32
_RPW = _NR_OUT // _NW
_CH = 96
_NCH = _RPW // _CH


def _row_table() -> np.ndarray:
    a = np.arange(_ROW).reshape(_ORD, _K, _K)
    perms = [np.rot90(np.roll(a, shift=r, axis=0), k=r, axes=(-2, -1)).reshape(_ROW)
             for r in range(4)]
    r_, ic_, y_, x_, h_ = np.meshgrid(
        np.arange(4), np.arange(_IC), np.arange(_K), np.arange(_K),
        np.arange(_ORD), indexing="ij")
    j = (h_ * _K + y_) * _K + x_
    src = np.stack(perms)[r_.ravel(), j.ravel()].reshape(j.shape)
    hp = src // (_K * _K)
    yp = (src % (_K * _K)) // _K
    xp = src % _K
    tab = (((ic_ * _K + yp) * _K + xp) * _ORD + hp).reshape(-1)
    return tab.astype(np.int32).reshape(_NW, _NCH, _CH)


_TAB = _row_table()

_MESH = plsc.VectorSubcoreMesh(core_axis_name="c", subcore_axis_name="s",
                               num_cores=2, num_subcores=16)


@functools.partial(
    pl.kernel,
    out_type=jax.ShapeDtypeStruct((_NR_OUT // _ORD, _ORD, _OC), jnp.float32),
    mesh=_MESH,
    scratch_types=[
        pltpu.VMEM((_NCH, _CH), jnp.int32),
        pltpu.VMEM((3, _CH, _OC), jnp.float32),
        pltpu.SemaphoreType.DMA,
        pltpu.SemaphoreType.DMA,
        pltpu.SemaphoreType.DMA,
        pltpu.SemaphoreType.DMA,
        pltpu.SemaphoreType.DMA,
        pltpu.SemaphoreType.DMA,
    ],
    compiler_params=pltpu.CompilerParams(needs_layout_passes=False),
)
def _bank(in3_hbm, tab_hbm, out3_hbm, idx_v, buf, *sems):
    in_hbm = in3_hbm.reshape(_NR_IN, _OC)
    out_hbm = out3_hbm.reshape(_NR_OUT, _OC)
    wid = lax.axis_index("s") * 2 + lax.axis_index("c")
    base = wid * _RPW
    pltpu.sync_copy(tab_hbm.at[wid], idx_v)
    gsems = sems[:3]
    osems = sems[3:]

    def gather(c):
        return pltpu.async_copy(in_hbm.at[idx_v.at[c]], buf.at[c % 3],
                                gsems[c % 3])

    def put(c):
        return pltpu.async_copy(buf.at[c % 3],
                                out_hbm.at[pl.ds(base + c * _CH, _CH)],
                                osems[c % 3])

    gh = {0: gather(0), 1: gather(1)}
    oh = {}
    for c in range(_NCH):
        gh[c].wait()
        oh[c] = put(c)
        n = c + 2
        if n < _NCH:
            if n - 3 >= 0:
                oh[n - 3].wait()
            gh[n] = gather(n)
    for c in range(_NCH - 3, _NCH):
        oh[c].wait()


def kernel(weight):
    in3 = weight.transpose(1, 3, 4, 2, 0).reshape(_NR_IN // _ORD, _ORD, _OC)
    out3 = _bank(in3, jnp.asarray(_TAB))
    out6 = out3.reshape(4, _IC, _K, _K, _ORD, _OC)
    return out6.transpose(5, 0, 1, 4, 2, 3)
